# initial kernel scaffold (unmeasured)
import jax
import jax.numpy as jnp
from jax import lax
from jax.experimental import pallas as pl
from jax.experimental.pallas import tpu as pltpu


def kernel(
    x,
):
    def body(*refs):
        pass

    out_shape = jax.ShapeDtypeStruct(..., jnp.float32)
    return pl.pallas_call(body, out_shape=out_shape)(...)


# baseline (device time: 11203 ns/iter reference)
import jax
import jax.numpy as jnp
from jax import lax
from jax.experimental import pallas as pl
from jax.experimental.pallas import tpu as pltpu

K = 8
NEG = float("-inf")


def _topk_desc(x, k):
    rows, n = x.shape
    col = lax.broadcasted_iota(jnp.int32, (rows, n), 1)
    vals = []
    cur = x
    for _ in range(k):
        m = jnp.max(cur, axis=1, keepdims=True)
        vals.append(m)
        eq = cur == m
        first = jnp.min(jnp.where(eq, col, n), axis=1, keepdims=True)
        cur = jnp.where(col == first, NEG, cur)
    return jnp.concatenate(vals, axis=1)


def kernel(x):
    rows, n = x.shape

    def body(x_ref, out_ref, comm_ref, send_sem, recv_sem):
        my_x = lax.axis_index("x")
        my_y = lax.axis_index("y")
        nbr = (1 - my_x, my_y)

        barrier_sem = pltpu.get_barrier_semaphore()
        pl.semaphore_signal(
            barrier_sem, inc=1, device_id=nbr,
            device_id_type=pl.DeviceIdType.MESH,
        )
        pl.semaphore_wait(barrier_sem, 1)

        local = _topk_desc(x_ref[:, :], K)
        comm_ref[0, :, :] = local

        rdma = pltpu.make_async_remote_copy(
            src_ref=comm_ref.at[0],
            dst_ref=comm_ref.at[1],
            send_sem=send_sem,
            recv_sem=recv_sem,
            device_id=nbr,
            device_id_type=pl.DeviceIdType.MESH,
        )
        rdma.start()
        rdma.wait()

        cand = jnp.concatenate([local, comm_ref[1, :, :]], axis=1)
        out_ref[:, :] = _topk_desc(cand, K)

    return pl.pallas_call(
        body,
        out_shape=jax.ShapeDtypeStruct((rows, K), jnp.float32),
        in_specs=[pl.BlockSpec(memory_space=pltpu.VMEM)],
        out_specs=pl.BlockSpec(memory_space=pltpu.VMEM),
        scratch_shapes=[
            pltpu.VMEM((2, rows, K), jnp.float32),
            pltpu.SemaphoreType.DMA,
            pltpu.SemaphoreType.DMA,
        ],
        compiler_params=pltpu.CompilerParams(collective_id=0),
    )(x)


# device time: 9621 ns/iter; 1.1644x vs baseline; 1.1644x over previous
import jax
import jax.numpy as jnp
from jax import lax
from jax.experimental import pallas as pl
from jax.experimental.pallas import tpu as pltpu

K = 8
NEG = float("-inf")


def _topk_desc_distinct(x, k):
    vals = []
    cur = x
    for _ in range(k):
        m = jnp.max(cur, axis=1, keepdims=True)
        vals.append(m)
        cur = jnp.where(cur == m, NEG, cur)
    return jnp.concatenate(vals, axis=1)


def _topk_desc_exact(x, k):
    rows, n = x.shape
    col = lax.broadcasted_iota(jnp.int32, (rows, n), 1)
    vals = []
    cur = x
    for _ in range(k):
        m = jnp.max(cur, axis=1, keepdims=True)
        vals.append(m)
        eq = cur == m
        first = jnp.min(jnp.where(eq, col, n), axis=1, keepdims=True)
        cur = jnp.where(col == first, NEG, cur)
    return jnp.concatenate(vals, axis=1)


def kernel(x):
    rows, n = x.shape

    def body(x_ref, out_ref, comm_ref, send_sem, recv_sem):
        my_x = lax.axis_index("x")
        my_y = lax.axis_index("y")
        nbr = (1 - my_x, my_y)

        barrier_sem = pltpu.get_barrier_semaphore()
        pl.semaphore_signal(
            barrier_sem, inc=1, device_id=nbr,
            device_id_type=pl.DeviceIdType.MESH,
        )

        xv = x_ref[:, :]
        col = lax.broadcasted_iota(jnp.int32, (rows, n), 1)
        bits = lax.bitcast_convert_type(xv, jnp.int32)
        bits = jnp.bitwise_or(jnp.bitwise_and(bits, jnp.int32(-1024)), col)
        xd = lax.bitcast_convert_type(bits, jnp.float32)

        local = _topk_desc_distinct(xd, K)
        comm_ref[0, :, :] = local

        pl.semaphore_wait(barrier_sem, 1)
        rdma = pltpu.make_async_remote_copy(
            src_ref=comm_ref.at[0],
            dst_ref=comm_ref.at[1],
            send_sem=send_sem,
            recv_sem=recv_sem,
            device_id=nbr,
            device_id_type=pl.DeviceIdType.MESH,
        )
        rdma.start()
        rdma.wait()

        cand = jnp.concatenate([local, comm_ref[1, :, :]], axis=1)
        out_ref[:, :] = _topk_desc_exact(cand, K)

    return pl.pallas_call(
        body,
        out_shape=jax.ShapeDtypeStruct((rows, K), jnp.float32),
        in_specs=[pl.BlockSpec(memory_space=pltpu.VMEM)],
        out_specs=pl.BlockSpec(memory_space=pltpu.VMEM),
        scratch_shapes=[
            pltpu.VMEM((2, rows, K), jnp.float32),
            pltpu.SemaphoreType.DMA,
            pltpu.SemaphoreType.DMA,
        ],
        compiler_params=pltpu.CompilerParams(collective_id=0),
    )(x)


# device time: 8580 ns/iter; 1.3057x vs baseline; 1.1213x over previous
import jax
import jax.numpy as jnp
from jax import lax
from jax.experimental import pallas as pl
from jax.experimental.pallas import tpu as pltpu

K = 8
NEG = float("-inf")


def _topk_desc_distinct(x, k):
    vals = []
    cur = x
    for _ in range(k):
        m = jnp.max(cur, axis=1, keepdims=True)
        vals.append(m)
        cur = jnp.where(cur == m, NEG, cur)
    return jnp.concatenate(vals, axis=1)


def _merge_sorted8(a, b):
    b_rev = jnp.concatenate(
        [b[:, i:i + 1] for i in range(K - 1, -1, -1)], axis=1
    )
    L = jnp.maximum(a, b_rev)
    hi = jnp.maximum(L[:, :4], L[:, 4:])
    lo = jnp.minimum(L[:, :4], L[:, 4:])
    L = jnp.concatenate([hi, lo], axis=1)
    parts = []
    for s in (0, 4):
        x0 = L[:, s:s + 2]
        x1 = L[:, s + 2:s + 4]
        parts += [jnp.maximum(x0, x1), jnp.minimum(x0, x1)]
    L = jnp.concatenate(parts, axis=1)
    parts = []
    for s in (0, 2, 4, 6):
        x0 = L[:, s:s + 1]
        x1 = L[:, s + 1:s + 2]
        parts += [jnp.maximum(x0, x1), jnp.minimum(x0, x1)]
    return jnp.concatenate(parts, axis=1)


def kernel(x):
    rows, n = x.shape

    def body(x_ref, out_ref, comm_ref, send_sem, recv_sem):
        my_x = lax.axis_index("x")
        my_y = lax.axis_index("y")
        nbr = (1 - my_x, my_y)

        barrier_sem = pltpu.get_barrier_semaphore()
        pl.semaphore_signal(
            barrier_sem, inc=1, device_id=nbr,
            device_id_type=pl.DeviceIdType.MESH,
        )

        xv = x_ref[:, :]
        col = lax.broadcasted_iota(jnp.int32, (rows, n), 1)
        bits = lax.bitcast_convert_type(xv, jnp.int32)
        bits = jnp.bitwise_or(jnp.bitwise_and(bits, jnp.int32(-1024)), col)
        xd = lax.bitcast_convert_type(bits, jnp.float32)

        local = _topk_desc_distinct(xd, K)
        comm_ref[0, :, :] = local

        pl.semaphore_wait(barrier_sem, 1)
        rdma = pltpu.make_async_remote_copy(
            src_ref=comm_ref.at[0],
            dst_ref=comm_ref.at[1],
            send_sem=send_sem,
            recv_sem=recv_sem,
            device_id=nbr,
            device_id_type=pl.DeviceIdType.MESH,
        )
        rdma.start()
        rdma.wait()

        out_ref[:, :] = _merge_sorted8(local, comm_ref[1, :, :])

    return pl.pallas_call(
        body,
        out_shape=jax.ShapeDtypeStruct((rows, K), jnp.float32),
        in_specs=[pl.BlockSpec(memory_space=pltpu.VMEM)],
        out_specs=pl.BlockSpec(memory_space=pltpu.VMEM),
        scratch_shapes=[
            pltpu.VMEM((2, rows, K), jnp.float32),
            pltpu.SemaphoreType.DMA,
            pltpu.SemaphoreType.DMA,
        ],
        compiler_params=pltpu.CompilerParams(collective_id=0),
    )(x)
